# baseline (speedup 1.0000x reference)
import jax
import jax.numpy as jnp
from jax.experimental import pallas as pl

N = 116
NP = 128
F = 220
NHID = 20
NHEADS = 8
BS = 2


def _gat_kernel(x_ref, w1_ref, asr1_ref, adt1_ref, b1_ref,
                w2_ref, as2_ref, ad2_ref, b2_ref, out_ref):
    row_i = jax.lax.broadcasted_iota(jnp.int32, (NP, NP), 0)
    col_j = jax.lax.broadcasted_iota(jnp.int32, (NP, NP), 1)
    eyef = jnp.where(row_i == col_j, 1.0, 0.0).astype(jnp.float32)
    ones_col = jnp.ones((NP, 1), dtype=jnp.bfloat16)

    w1 = w1_ref[...]
    asr1 = asr1_ref[...]
    adt1 = adt1_ref[...]
    w2 = w2_ref[...]
    f32 = jnp.float32

    def leaky(v):
        return jnp.maximum(v, 0.2 * v)

    for s in range(BS):
        x = x_ref[s]
        mean = jnp.sum(x, axis=1, keepdims=True) * (1.0 / F)
        xc = x - mean
        cov = jax.lax.dot_general(xc, xc, (((1,), (1,)), ((), ())),
                                  preferred_element_type=f32)
        d = jnp.sqrt(jnp.sum(xc * xc, axis=1, keepdims=True))
        maskf = jnp.maximum(
            jnp.where(cov > 0.5 * d * jnp.transpose(d), 1.0, 0.0), eyef)

        h1 = jnp.dot(x, w1, preferred_element_type=f32)
        h1b = h1.astype(jnp.bfloat16)
        a_srcT = jax.lax.dot_general(asr1, h1, (((0,), (1,)), ((), ())),
                                     preferred_element_type=f32)
        a_dst = jnp.dot(h1, adt1, preferred_element_type=f32)
        msgs = []
        for hd in range(NHEADS):
            dst = a_dst[:, hd:hd + 1]
            src = a_srcT[hd:hd + 1, :]
            a_max = jnp.max(src, axis=1, keepdims=True)
            bound = leaky(a_max + dst)
            e = jnp.exp(leaky(dst + src) - bound) * maskf
            eb = e.astype(jnp.bfloat16)
            num = jax.lax.dot_general(eb, h1b[:, hd * NHID:(hd + 1) * NHID],
                                      (((1,), (0,)), ((), ())),
                                      preferred_element_type=f32)
            den = jax.lax.dot_general(eb, ones_col, (((1,), (0,)), ((), ())),
                                      preferred_element_type=f32)
            msgs.append(num * jax.lax.reciprocal(den))
        out1 = jnp.concatenate(msgs, axis=1) + b1_ref[...]
        out1 = jnp.where(out1 > 0, out1, jnp.exp(out1) - 1.0)

        h2 = jnp.dot(out1, w2, preferred_element_type=f32)
        a2sT = jax.lax.dot_general(as2_ref[...], h2, (((1,), (1,)), ((), ())),
                                   preferred_element_type=f32)
        a2d = jax.lax.dot_general(h2, ad2_ref[...], (((1,), (1,)), ((), ())),
                                  preferred_element_type=f32)
        bound2 = leaky(jnp.max(a2sT, axis=1, keepdims=True) + a2d)
        e2 = jnp.exp(leaky(a2d + a2sT) - bound2) * maskf
        e2b = e2.astype(jnp.bfloat16)
        num2 = jax.lax.dot_general(e2b, h2.astype(jnp.bfloat16),
                                   (((1,), (0,)), ((), ())),
                                   preferred_element_type=f32)
        den2 = jax.lax.dot_general(e2b, ones_col, (((1,), (0,)), ((), ())),
                                   preferred_element_type=f32)
        out_ref[s, :, :] = num2 * jax.lax.reciprocal(den2) + b2_ref[...]


def _mlp_kernel(flat_ref, l1w_ref, l1b_ref, g1_ref, bt1_ref,
                l2w_ref, l2b_ref, g2_ref, bt2_ref, l3w_ref, l3b_ref,
                probs_ref, block_ref):
    inv = jnp.float32(1.0 / (1.0 + 1e-5) ** 0.5)
    h = jnp.dot(flat_ref[...], l1w_ref[...]) + l1b_ref[...]
    h = h * (g1_ref[...] * inv) + bt1_ref[...]
    blk = jnp.dot(h, l2w_ref[...]) + l2b_ref[...]
    blk = blk * (g2_ref[...] * inv) + bt2_ref[...]
    block_ref[...] = blk
    lg = jnp.dot(blk, l3w_ref[...]) + l3b_ref[...]
    m = jnp.max(lg, axis=1, keepdims=True)
    e = jnp.exp(lg - m)
    probs_ref[...] = e / jnp.sum(e, axis=1, keepdims=True)


@jax.jit
def kernel(input, W1, as1, ad1, b1, W2, as2, ad2, b2, l1_w, l1_b,
           bn1_g, bn1_b, l2_w, l2_b, bn2_g, bn2_b, l3_w, l3_b):
    B = input.shape[0]
    xp = jnp.pad(input, ((0, 0), (0, NP - N), (0, 0)))
    eye8 = jnp.eye(NHEADS, dtype=jnp.float32)
    asr1 = (eye8[:, None, :] * as1[:, :, None]).reshape(NHEADS * NHID, NHEADS)
    adt1 = (eye8[:, None, :] * ad1[:, :, None]).reshape(NHEADS * NHID, NHEADS)

    full = lambda shp: pl.BlockSpec(shp, lambda i: (0,) * len(shp))
    gat = pl.pallas_call(
        _gat_kernel,
        grid=(B // BS,),
        in_specs=[
            pl.BlockSpec((BS, NP, F), lambda i: (i, 0, 0)),
            full((F, NHEADS * NHID)),
            full((NHEADS * NHID, NHEADS)),
            full((NHEADS * NHID, NHEADS)),
            full((1, NHEADS * NHID)),
            full((NHEADS * NHID, NHID)),
            full((1, NHID)),
            full((1, NHID)),
            full((1, NHID)),
        ],
        out_specs=pl.BlockSpec((BS, NP, NHID), lambda i: (i, 0, 0)),
        out_shape=jax.ShapeDtypeStruct((B, NP, NHID), jnp.float32),
    )(xp, W1, asr1, adt1, b1.reshape(1, -1), W2,
      as2.reshape(1, -1), ad2.reshape(1, -1), b2.reshape(1, -1))

    flat = gat[:, :N, :].reshape(B, N * NHID)
    probs, block = pl.pallas_call(
        _mlp_kernel,
        out_shape=(jax.ShapeDtypeStruct((B, 2), jnp.float32),
                   jax.ShapeDtypeStruct((B, 256), jnp.float32)),
    )(flat, l1_w, l1_b.reshape(1, -1), bn1_g.reshape(1, -1),
      bn1_b.reshape(1, -1), l2_w, l2_b.reshape(1, -1),
      bn2_g.reshape(1, -1), bn2_b.reshape(1, -1), l3_w, l3_b.reshape(1, -1))
    return probs, block


# speedup vs baseline: 1.4795x; 1.4795x over previous
import jax
import jax.numpy as jnp
from jax.experimental import pallas as pl

N = 116
NP = 128
F = 220
NHID = 20
NHEADS = 8
BS = 4


def _gat_kernel(xt_ref, w1t_ref, asr_ref, adt_ref, b1_ref,
                w2t_ref, as2_ref, ad2_ref, b2_ref, out_ref):
    row_i = jax.lax.broadcasted_iota(jnp.int32, (NP, NP), 0)
    col_j = jax.lax.broadcasted_iota(jnp.int32, (NP, NP), 1)
    eyef = jnp.where(row_i == col_j, 1.0, 0.0).astype(jnp.float32)
    ones_row = jnp.ones((1, NP), dtype=jnp.bfloat16)
    ones_fr = jnp.ones((1, F), dtype=jnp.float32)

    w1t = w1t_ref[...]
    asr = asr_ref[...]
    adt = adt_ref[...]
    w2t = w2t_ref[...]
    b1w = b1_ref[...] * jnp.ones((1, NP), dtype=jnp.float32)
    b2w = b2_ref[...] * jnp.ones((1, NP), dtype=jnp.float32)
    f32 = jnp.float32

    def leaky(v):
        return jnp.maximum(v, 0.2 * v)

    maskb, h1tb, a_src, a_dst, bound = [], [], [], [], []
    for s in range(BS):
        xt = xt_ref[s]
        mean = jnp.dot(ones_fr, xt, preferred_element_type=f32) * (1.0 / F)
        xct = xt - mean
        r = jnp.dot(ones_fr, xct * xct, preferred_element_type=f32)
        xnt = xct * jax.lax.rsqrt(jnp.maximum(r, 1e-30))
        corr = jax.lax.dot_general(xnt, xnt, (((0,), (0,)), ((), ())),
                                   preferred_element_type=f32)
        maskb.append(
            jnp.maximum(jnp.where(corr > 0.5, 1.0, 0.0), eyef)
            .astype(jnp.bfloat16))
        h1t = jnp.dot(w1t, xt, preferred_element_type=f32)
        h1tb.append(h1t.astype(jnp.bfloat16))
        asd = jax.lax.dot_general(asr, h1t, (((1,), (0,)), ((), ())),
                                  preferred_element_type=f32)
        adst = jax.lax.dot_general(adt, h1t, (((1,), (0,)), ((), ())),
                                   preferred_element_type=f32)
        a_src.append(asd)
        a_dst.append(adst)
        bound.append(leaky(jnp.max(asd, axis=1, keepdims=True) + adst))

    msgs = [[] for _ in range(BS)]
    for s in range(BS):
        for hd in range(NHEADS):
            src_c = jnp.transpose(a_src[s][hd:hd + 1, :])
            e = jnp.exp(leaky(src_c + a_dst[s][hd:hd + 1, :])
                        - bound[s][hd:hd + 1, :])
            eb = e.astype(jnp.bfloat16) * maskb[s]
            h1e = jnp.concatenate(
                [h1tb[s][hd * NHID:(hd + 1) * NHID, :], ones_row], axis=0)
            nd = jnp.dot(h1e, eb, preferred_element_type=f32)
            msgs[s].append(nd[:NHID, :]
                           * jax.lax.reciprocal(nd[NHID:NHID + 1, :]))

    for s in range(BS):
        out1 = jnp.concatenate(msgs[s], axis=0) + b1w
        out1 = jnp.where(out1 > 0, out1, jnp.exp(out1) - 1.0)
        h2t = jnp.dot(w2t, out1, preferred_element_type=f32)
        h2tb = h2t.astype(jnp.bfloat16)
        a2s = jnp.dot(as2_ref[...], h2t, preferred_element_type=f32)
        a2d = jnp.dot(ad2_ref[...], h2t, preferred_element_type=f32)
        bound2 = leaky(jnp.max(a2s, axis=1, keepdims=True) + a2d)
        e2 = jnp.exp(leaky(jnp.transpose(a2s) + a2d) - bound2)
        e2b = e2.astype(jnp.bfloat16) * maskb[s]
        h2e = jnp.concatenate([h2tb, ones_row], axis=0)
        nd2 = jnp.dot(h2e, e2b, preferred_element_type=f32)
        out_ref[s, :, :] = (nd2[:NHID, :]
                            * jax.lax.reciprocal(nd2[NHID:NHID + 1, :]) + b2w)


def _mlp_kernel(flat_ref, l1w_ref, l1b_ref, g1_ref, bt1_ref,
                l2w_ref, l2b_ref, g2_ref, bt2_ref, l3w_ref, l3b_ref,
                probs_ref, block_ref):
    inv = jnp.float32(1.0 / (1.0 + 1e-5) ** 0.5)
    h = jnp.dot(flat_ref[...], l1w_ref[...]) + l1b_ref[...]
    h = h * (g1_ref[...] * inv) + bt1_ref[...]
    blk = jnp.dot(h, l2w_ref[...]) + l2b_ref[...]
    blk = blk * (g2_ref[...] * inv) + bt2_ref[...]
    block_ref[...] = blk
    lg = jnp.dot(blk, l3w_ref[...]) + l3b_ref[...]
    m = jnp.max(lg, axis=1, keepdims=True)
    e = jnp.exp(lg - m)
    probs_ref[...] = e / jnp.sum(e, axis=1, keepdims=True)


@jax.jit
def kernel(input, W1, as1, ad1, b1, W2, as2, ad2, b2, l1_w, l1_b,
           bn1_g, bn1_b, l2_w, l2_b, bn2_g, bn2_b, l3_w, l3_b):
    B = input.shape[0]
    xpt = jnp.pad(input, ((0, 0), (0, NP - N), (0, 0))).transpose(0, 2, 1)
    eye8 = jnp.eye(NHEADS, dtype=jnp.float32)
    asr = (eye8[:, None, :] * as1[:, :, None]).reshape(
        NHEADS * NHID, NHEADS).T
    adt = (eye8[:, None, :] * ad1[:, :, None]).reshape(
        NHEADS * NHID, NHEADS).T
    l1p = l1_w.reshape(N, NHID, -1).transpose(1, 0, 2).reshape(N * NHID, -1)

    full = lambda shp: pl.BlockSpec(shp, lambda i: (0,) * len(shp))
    gat_t = pl.pallas_call(
        _gat_kernel,
        grid=(B // BS,),
        in_specs=[
            pl.BlockSpec((BS, F, NP), lambda i: (i, 0, 0)),
            full((NHEADS * NHID, F)),
            full((NHEADS, NHEADS * NHID)),
            full((NHEADS, NHEADS * NHID)),
            full((NHEADS * NHID, 1)),
            full((NHID, NHEADS * NHID)),
            full((1, NHID)),
            full((1, NHID)),
            full((NHID, 1)),
        ],
        out_specs=pl.BlockSpec((BS, NHID, NP), lambda i: (i, 0, 0)),
        out_shape=jax.ShapeDtypeStruct((B, NHID, NP), jnp.float32),
    )(xpt, W1.T, asr, adt, b1.reshape(-1, 1), W2.T,
      as2, ad2, b2.reshape(-1, 1))

    flat = gat_t[:, :, :N].reshape(B, NHID * N)
    probs, block = pl.pallas_call(
        _mlp_kernel,
        out_shape=(jax.ShapeDtypeStruct((B, 2), jnp.float32),
                   jax.ShapeDtypeStruct((B, 256), jnp.float32)),
    )(flat, l1p, l1_b.reshape(1, -1), bn1_g.reshape(1, -1),
      bn1_b.reshape(1, -1), l2_w, l2_b.reshape(1, -1),
      bn2_g.reshape(1, -1), bn2_b.reshape(1, -1), l3_w, l3_b.reshape(1, -1))
    return probs, block


# speedup vs baseline: 1.5680x; 1.0598x over previous
import jax
import jax.numpy as jnp
from jax.experimental import pallas as pl

N = 116
NP = 128
F = 220
NHID = 20
NHEADS = 8
BS = 8


def _gat_kernel(xt_ref, w1t_ref, asr_ref, adt_ref, b1_ref,
                w2t_ref, as2_ref, ad2_ref, b2_ref, out_ref):
    row_i = jax.lax.broadcasted_iota(jnp.int32, (NP, NP), 0)
    col_j = jax.lax.broadcasted_iota(jnp.int32, (NP, NP), 1)
    eyef = jnp.where(row_i == col_j, 1.0, 0.0).astype(jnp.float32)
    ones_row = jnp.ones((1, NP), dtype=jnp.bfloat16)
    ones_fr = jnp.ones((1, F), dtype=jnp.float32)

    w1t = w1t_ref[...]
    asr = asr_ref[...]
    adt = adt_ref[...]
    w2t = w2t_ref[...]
    b1w = b1_ref[...] * jnp.ones((1, NP), dtype=jnp.float32)
    b2w = b2_ref[...] * jnp.ones((1, NP), dtype=jnp.float32)
    f32 = jnp.float32

    def leaky(v):
        return jnp.maximum(v, 0.2 * v)

    maskb, h1tb, a_src, a_dst, bound = [], [], [], [], []
    for s in range(BS):
        xt = xt_ref[s]
        mean = jnp.dot(ones_fr, xt, preferred_element_type=f32) * (1.0 / F)
        xct = xt - mean
        r = jnp.dot(ones_fr, xct * xct, preferred_element_type=f32)
        xnt = xct * jax.lax.rsqrt(jnp.maximum(r, 1e-30))
        corr = jax.lax.dot_general(xnt, xnt, (((0,), (0,)), ((), ())),
                                   preferred_element_type=f32)
        maskb.append(
            jnp.maximum(jnp.where(corr > 0.5, 1.0, 0.0), eyef)
            .astype(jnp.bfloat16))
        h1t = jnp.dot(w1t, xt, preferred_element_type=f32)
        h1tb.append(h1t.astype(jnp.bfloat16))
        asd = jax.lax.dot_general(asr, h1t, (((1,), (0,)), ((), ())),
                                  preferred_element_type=f32)
        adst = jax.lax.dot_general(adt, h1t, (((1,), (0,)), ((), ())),
                                   preferred_element_type=f32)
        a_src.append(asd)
        a_dst.append(adst)
        bound.append(leaky(jnp.max(asd, axis=1, keepdims=True) + adst))

    msgs = [[] for _ in range(BS)]
    for s in range(BS):
        for hd in range(NHEADS):
            src_c = jnp.transpose(a_src[s][hd:hd + 1, :])
            e = jnp.exp(leaky(src_c + a_dst[s][hd:hd + 1, :])
                        - bound[s][hd:hd + 1, :])
            eb = e.astype(jnp.bfloat16) * maskb[s]
            h1e = jnp.concatenate(
                [h1tb[s][hd * NHID:(hd + 1) * NHID, :], ones_row], axis=0)
            nd = jnp.dot(h1e, eb, preferred_element_type=f32)
            msgs[s].append(nd[:NHID, :]
                           * jax.lax.reciprocal(nd[NHID:NHID + 1, :]))

    for s in range(BS):
        out1 = jnp.concatenate(msgs[s], axis=0) + b1w
        out1 = jnp.where(out1 > 0, out1, jnp.exp(out1) - 1.0)
        h2t = jnp.dot(w2t, out1, preferred_element_type=f32)
        h2tb = h2t.astype(jnp.bfloat16)
        a2s = jnp.dot(as2_ref[...], h2t, preferred_element_type=f32)
        a2d = jnp.dot(ad2_ref[...], h2t, preferred_element_type=f32)
        bound2 = leaky(jnp.max(a2s, axis=1, keepdims=True) + a2d)
        e2 = jnp.exp(leaky(jnp.transpose(a2s) + a2d) - bound2)
        e2b = e2.astype(jnp.bfloat16) * maskb[s]
        h2e = jnp.concatenate([h2tb, ones_row], axis=0)
        nd2 = jnp.dot(h2e, e2b, preferred_element_type=f32)
        out_ref[s, :, :] = (nd2[:NHID, :]
                            * jax.lax.reciprocal(nd2[NHID:NHID + 1, :]) + b2w)


def _mlp_kernel(flat_ref, l1w_ref, l1b_ref, g1_ref, bt1_ref,
                l2w_ref, l2b_ref, g2_ref, bt2_ref, l3w_ref, l3b_ref,
                probs_ref, block_ref):
    inv = jnp.float32(1.0 / (1.0 + 1e-5) ** 0.5)
    h = jnp.dot(flat_ref[...], l1w_ref[...]) + l1b_ref[...]
    h = h * (g1_ref[...] * inv) + bt1_ref[...]
    blk = jnp.dot(h, l2w_ref[...]) + l2b_ref[...]
    blk = blk * (g2_ref[...] * inv) + bt2_ref[...]
    block_ref[...] = blk
    lg = jnp.dot(blk, l3w_ref[...]) + l3b_ref[...]
    m = jnp.max(lg, axis=1, keepdims=True)
    e = jnp.exp(lg - m)
    probs_ref[...] = e / jnp.sum(e, axis=1, keepdims=True)


@jax.jit
def kernel(input, W1, as1, ad1, b1, W2, as2, ad2, b2, l1_w, l1_b,
           bn1_g, bn1_b, l2_w, l2_b, bn2_g, bn2_b, l3_w, l3_b):
    B = input.shape[0]
    xpt = jnp.pad(input, ((0, 0), (0, NP - N), (0, 0))).transpose(0, 2, 1)
    eye8 = jnp.eye(NHEADS, dtype=jnp.float32)
    asr = (eye8[:, None, :] * as1[:, :, None]).reshape(
        NHEADS * NHID, NHEADS).T
    adt = (eye8[:, None, :] * ad1[:, :, None]).reshape(
        NHEADS * NHID, NHEADS).T
    l1p = l1_w.reshape(N, NHID, -1).transpose(1, 0, 2).reshape(N * NHID, -1)

    full = lambda shp: pl.BlockSpec(shp, lambda i: (0,) * len(shp))
    gat_t = pl.pallas_call(
        _gat_kernel,
        grid=(B // BS,),
        in_specs=[
            pl.BlockSpec((BS, F, NP), lambda i: (i, 0, 0)),
            full((NHEADS * NHID, F)),
            full((NHEADS, NHEADS * NHID)),
            full((NHEADS, NHEADS * NHID)),
            full((NHEADS * NHID, 1)),
            full((NHID, NHEADS * NHID)),
            full((1, NHID)),
            full((1, NHID)),
            full((NHID, 1)),
        ],
        out_specs=pl.BlockSpec((BS, NHID, NP), lambda i: (i, 0, 0)),
        out_shape=jax.ShapeDtypeStruct((B, NHID, NP), jnp.float32),
    )(xpt, W1.T, asr, adt, b1.reshape(-1, 1), W2.T,
      as2, ad2, b2.reshape(-1, 1))

    flat = gat_t[:, :, :N].reshape(B, NHID * N)
    probs, block = pl.pallas_call(
        _mlp_kernel,
        out_shape=(jax.ShapeDtypeStruct((B, 2), jnp.float32),
                   jax.ShapeDtypeStruct((B, 256), jnp.float32)),
    )(flat, l1p, l1_b.reshape(1, -1), bn1_g.reshape(1, -1),
      bn1_b.reshape(1, -1), l2_w, l2_b.reshape(1, -1),
      bn2_g.reshape(1, -1), bn2_b.reshape(1, -1), l3_w, l3_b.reshape(1, -1))
    return probs, block


# speedup vs baseline: 1.6367x; 1.0438x over previous
import jax
import jax.numpy as jnp
from jax.experimental import pallas as pl

N = 116
NP = 128
F = 220
NHID = 20
NHEADS = 8
BS = 16


def _gat_kernel(xt_ref, w1t_ref, asr_ref, adt_ref, b1_ref,
                w2t_ref, as2_ref, ad2_ref, b2_ref, out_ref):
    row_i = jax.lax.broadcasted_iota(jnp.int32, (NP, NP), 0)
    col_j = jax.lax.broadcasted_iota(jnp.int32, (NP, NP), 1)
    eyef = jnp.where(row_i == col_j, 1.0, 0.0).astype(jnp.float32)
    ones_row = jnp.ones((1, NP), dtype=jnp.bfloat16)
    ones_fr = jnp.ones((1, F), dtype=jnp.float32)

    w1t = w1t_ref[...]
    asr = asr_ref[...]
    adt = adt_ref[...]
    w2t = w2t_ref[...]
    b1w = b1_ref[...] * jnp.ones((1, NP), dtype=jnp.float32)
    b2w = b2_ref[...] * jnp.ones((1, NP), dtype=jnp.float32)
    f32 = jnp.float32

    def leaky(v):
        return jnp.maximum(v, 0.2 * v)

    maskb, h1tb, a_src, a_dst, bound = [], [], [], [], []
    for s in range(BS):
        xt = xt_ref[s]
        mean = jnp.dot(ones_fr, xt, preferred_element_type=f32) * (1.0 / F)
        xct = xt - mean
        r = jnp.dot(ones_fr, xct * xct, preferred_element_type=f32)
        xnt = xct * jax.lax.rsqrt(jnp.maximum(r, 1e-30))
        corr = jax.lax.dot_general(xnt, xnt, (((0,), (0,)), ((), ())),
                                   preferred_element_type=f32)
        maskb.append(
            jnp.maximum(jnp.where(corr > 0.5, 1.0, 0.0), eyef)
            .astype(jnp.bfloat16))
        h1t = jnp.dot(w1t, xt, preferred_element_type=f32)
        h1tb.append(h1t.astype(jnp.bfloat16))
        asd = jax.lax.dot_general(asr, h1t, (((1,), (0,)), ((), ())),
                                  preferred_element_type=f32)
        adst = jax.lax.dot_general(adt, h1t, (((1,), (0,)), ((), ())),
                                   preferred_element_type=f32)
        a_src.append(asd)
        a_dst.append(adst)
        bound.append(leaky(jnp.max(asd, axis=1, keepdims=True) + adst))

    msgs = [[] for _ in range(BS)]
    for s in range(BS):
        for hd in range(NHEADS):
            src_c = jnp.transpose(a_src[s][hd:hd + 1, :])
            e = jnp.exp(leaky(src_c + a_dst[s][hd:hd + 1, :])
                        - bound[s][hd:hd + 1, :])
            eb = e.astype(jnp.bfloat16) * maskb[s]
            h1e = jnp.concatenate(
                [h1tb[s][hd * NHID:(hd + 1) * NHID, :], ones_row], axis=0)
            nd = jnp.dot(h1e, eb, preferred_element_type=f32)
            msgs[s].append(nd[:NHID, :]
                           * jax.lax.reciprocal(nd[NHID:NHID + 1, :]))

    for s in range(BS):
        out1 = jnp.concatenate(msgs[s], axis=0) + b1w
        out1 = jnp.where(out1 > 0, out1, jnp.exp(out1) - 1.0)
        h2t = jnp.dot(w2t, out1, preferred_element_type=f32)
        h2tb = h2t.astype(jnp.bfloat16)
        a2s = jnp.dot(as2_ref[...], h2t, preferred_element_type=f32)
        a2d = jnp.dot(ad2_ref[...], h2t, preferred_element_type=f32)
        bound2 = leaky(jnp.max(a2s, axis=1, keepdims=True) + a2d)
        e2 = jnp.exp(leaky(jnp.transpose(a2s) + a2d) - bound2)
        e2b = e2.astype(jnp.bfloat16) * maskb[s]
        h2e = jnp.concatenate([h2tb, ones_row], axis=0)
        nd2 = jnp.dot(h2e, e2b, preferred_element_type=f32)
        out_ref[s, :, :] = (nd2[:NHID, :]
                            * jax.lax.reciprocal(nd2[NHID:NHID + 1, :]) + b2w)


def _mlp_kernel(flat_ref, l1w_ref, l1b_ref, g1_ref, bt1_ref,
                l2w_ref, l2b_ref, g2_ref, bt2_ref, l3w_ref, l3b_ref,
                probs_ref, block_ref):
    inv = jnp.float32(1.0 / (1.0 + 1e-5) ** 0.5)
    h = jnp.dot(flat_ref[...], l1w_ref[...]) + l1b_ref[...]
    h = h * (g1_ref[...] * inv) + bt1_ref[...]
    blk = jnp.dot(h, l2w_ref[...]) + l2b_ref[...]
    blk = blk * (g2_ref[...] * inv) + bt2_ref[...]
    block_ref[...] = blk
    lg = jnp.dot(blk, l3w_ref[...]) + l3b_ref[...]
    m = jnp.max(lg, axis=1, keepdims=True)
    e = jnp.exp(lg - m)
    probs_ref[...] = e / jnp.sum(e, axis=1, keepdims=True)


@jax.jit
def kernel(input, W1, as1, ad1, b1, W2, as2, ad2, b2, l1_w, l1_b,
           bn1_g, bn1_b, l2_w, l2_b, bn2_g, bn2_b, l3_w, l3_b):
    B = input.shape[0]
    xpt = jnp.pad(input, ((0, 0), (0, NP - N), (0, 0))).transpose(0, 2, 1)
    eye8 = jnp.eye(NHEADS, dtype=jnp.float32)
    asr = (eye8[:, None, :] * as1[:, :, None]).reshape(
        NHEADS * NHID, NHEADS).T
    adt = (eye8[:, None, :] * ad1[:, :, None]).reshape(
        NHEADS * NHID, NHEADS).T
    l1p = l1_w.reshape(N, NHID, -1).transpose(1, 0, 2).reshape(N * NHID, -1)

    full = lambda shp: pl.BlockSpec(shp, lambda i: (0,) * len(shp))
    gat_t = pl.pallas_call(
        _gat_kernel,
        grid=(B // BS,),
        in_specs=[
            pl.BlockSpec((BS, F, NP), lambda i: (i, 0, 0)),
            full((NHEADS * NHID, F)),
            full((NHEADS, NHEADS * NHID)),
            full((NHEADS, NHEADS * NHID)),
            full((NHEADS * NHID, 1)),
            full((NHID, NHEADS * NHID)),
            full((1, NHID)),
            full((1, NHID)),
            full((NHID, 1)),
        ],
        out_specs=pl.BlockSpec((BS, NHID, NP), lambda i: (i, 0, 0)),
        out_shape=jax.ShapeDtypeStruct((B, NHID, NP), jnp.float32),
    )(xpt, W1.T, asr, adt, b1.reshape(-1, 1), W2.T,
      as2, ad2, b2.reshape(-1, 1))

    flat = gat_t[:, :, :N].reshape(B, NHID * N)
    probs, block = pl.pallas_call(
        _mlp_kernel,
        out_shape=(jax.ShapeDtypeStruct((B, 2), jnp.float32),
                   jax.ShapeDtypeStruct((B, 256), jnp.float32)),
    )(flat, l1p, l1_b.reshape(1, -1), bn1_g.reshape(1, -1),
      bn1_b.reshape(1, -1), l2_w, l2_b.reshape(1, -1),
      bn2_g.reshape(1, -1), bn2_b.reshape(1, -1), l3_w, l3_b.reshape(1, -1))
    return probs, block


# speedup vs baseline: 1.6528x; 1.0099x over previous
import jax
import jax.numpy as jnp
from jax.experimental import pallas as pl

N = 116
NP = 128
F = 220
NHID = 20
NHEADS = 8
BS = 32


def _gat_kernel(xt_ref, w1t_ref, asr_ref, adt_ref, b1_ref,
                w2t_ref, as2_ref, ad2_ref, b2_ref, out_ref):
    row_i = jax.lax.broadcasted_iota(jnp.int32, (NP, NP), 0)
    col_j = jax.lax.broadcasted_iota(jnp.int32, (NP, NP), 1)
    eyef = jnp.where(row_i == col_j, 1.0, 0.0).astype(jnp.float32)
    ones_row = jnp.ones((1, NP), dtype=jnp.bfloat16)
    ones_fr = jnp.ones((1, F), dtype=jnp.float32)

    w1t = w1t_ref[...]
    asr = asr_ref[...]
    adt = adt_ref[...]
    w2t = w2t_ref[...]
    b1w = b1_ref[...] * jnp.ones((1, NP), dtype=jnp.float32)
    b2w = b2_ref[...] * jnp.ones((1, NP), dtype=jnp.float32)
    f32 = jnp.float32

    def leaky(v):
        return jnp.maximum(v, 0.2 * v)

    maskb, h1tb, a_src, a_dst, bound = [], [], [], [], []
    for s in range(BS):
        xt = xt_ref[s]
        mean = jnp.dot(ones_fr, xt, preferred_element_type=f32) * (1.0 / F)
        xct = xt - mean
        r = jnp.dot(ones_fr, xct * xct, preferred_element_type=f32)
        xnt = xct * jax.lax.rsqrt(jnp.maximum(r, 1e-30))
        corr = jax.lax.dot_general(xnt, xnt, (((0,), (0,)), ((), ())),
                                   preferred_element_type=f32)
        maskb.append(
            jnp.maximum(jnp.where(corr > 0.5, 1.0, 0.0), eyef)
            .astype(jnp.bfloat16))
        h1t = jnp.dot(w1t, xt, preferred_element_type=f32)
        h1tb.append(h1t.astype(jnp.bfloat16))
        asd = jax.lax.dot_general(asr, h1t, (((1,), (0,)), ((), ())),
                                  preferred_element_type=f32)
        adst = jax.lax.dot_general(adt, h1t, (((1,), (0,)), ((), ())),
                                   preferred_element_type=f32)
        a_src.append(asd)
        a_dst.append(adst)
        bound.append(leaky(jnp.max(asd, axis=1, keepdims=True) + adst))

    msgs = [[] for _ in range(BS)]
    for s in range(BS):
        for hd in range(NHEADS):
            src_c = jnp.transpose(a_src[s][hd:hd + 1, :])
            e = jnp.exp(leaky(src_c + a_dst[s][hd:hd + 1, :])
                        - bound[s][hd:hd + 1, :])
            eb = e.astype(jnp.bfloat16) * maskb[s]
            h1e = jnp.concatenate(
                [h1tb[s][hd * NHID:(hd + 1) * NHID, :], ones_row], axis=0)
            nd = jnp.dot(h1e, eb, preferred_element_type=f32)
            msgs[s].append(nd[:NHID, :]
                           * jax.lax.reciprocal(nd[NHID:NHID + 1, :]))

    for s in range(BS):
        out1 = jnp.concatenate(msgs[s], axis=0) + b1w
        out1 = jnp.where(out1 > 0, out1, jnp.exp(out1) - 1.0)
        h2t = jnp.dot(w2t, out1, preferred_element_type=f32)
        h2tb = h2t.astype(jnp.bfloat16)
        a2s = jnp.dot(as2_ref[...], h2t, preferred_element_type=f32)
        a2d = jnp.dot(ad2_ref[...], h2t, preferred_element_type=f32)
        bound2 = leaky(jnp.max(a2s, axis=1, keepdims=True) + a2d)
        e2 = jnp.exp(leaky(jnp.transpose(a2s) + a2d) - bound2)
        e2b = e2.astype(jnp.bfloat16) * maskb[s]
        h2e = jnp.concatenate([h2tb, ones_row], axis=0)
        nd2 = jnp.dot(h2e, e2b, preferred_element_type=f32)
        out_ref[s, :, :] = (nd2[:NHID, :]
                            * jax.lax.reciprocal(nd2[NHID:NHID + 1, :]) + b2w)


def _mlp_kernel(flat_ref, l1w_ref, l1b_ref, g1_ref, bt1_ref,
                l2w_ref, l2b_ref, g2_ref, bt2_ref, l3w_ref, l3b_ref,
                probs_ref, block_ref):
    inv = jnp.float32(1.0 / (1.0 + 1e-5) ** 0.5)
    h = jnp.dot(flat_ref[...], l1w_ref[...]) + l1b_ref[...]
    h = h * (g1_ref[...] * inv) + bt1_ref[...]
    blk = jnp.dot(h, l2w_ref[...]) + l2b_ref[...]
    blk = blk * (g2_ref[...] * inv) + bt2_ref[...]
    block_ref[...] = blk
    lg = jnp.dot(blk, l3w_ref[...]) + l3b_ref[...]
    m = jnp.max(lg, axis=1, keepdims=True)
    e = jnp.exp(lg - m)
    probs_ref[...] = e / jnp.sum(e, axis=1, keepdims=True)


@jax.jit
def kernel(input, W1, as1, ad1, b1, W2, as2, ad2, b2, l1_w, l1_b,
           bn1_g, bn1_b, l2_w, l2_b, bn2_g, bn2_b, l3_w, l3_b):
    B = input.shape[0]
    xpt = jnp.pad(input, ((0, 0), (0, NP - N), (0, 0))).transpose(0, 2, 1)
    eye8 = jnp.eye(NHEADS, dtype=jnp.float32)
    asr = (eye8[:, None, :] * as1[:, :, None]).reshape(
        NHEADS * NHID, NHEADS).T
    adt = (eye8[:, None, :] * ad1[:, :, None]).reshape(
        NHEADS * NHID, NHEADS).T
    l1p = l1_w.reshape(N, NHID, -1).transpose(1, 0, 2).reshape(N * NHID, -1)

    full = lambda shp: pl.BlockSpec(shp, lambda i: (0,) * len(shp))
    gat_t = pl.pallas_call(
        _gat_kernel,
        grid=(B // BS,),
        in_specs=[
            pl.BlockSpec((BS, F, NP), lambda i: (i, 0, 0)),
            full((NHEADS * NHID, F)),
            full((NHEADS, NHEADS * NHID)),
            full((NHEADS, NHEADS * NHID)),
            full((NHEADS * NHID, 1)),
            full((NHID, NHEADS * NHID)),
            full((1, NHID)),
            full((1, NHID)),
            full((NHID, 1)),
        ],
        out_specs=pl.BlockSpec((BS, NHID, NP), lambda i: (i, 0, 0)),
        out_shape=jax.ShapeDtypeStruct((B, NHID, NP), jnp.float32),
    )(xpt, W1.T, asr, adt, b1.reshape(-1, 1), W2.T,
      as2, ad2, b2.reshape(-1, 1))

    flat = gat_t[:, :, :N].reshape(B, NHID * N)
    probs, block = pl.pallas_call(
        _mlp_kernel,
        out_shape=(jax.ShapeDtypeStruct((B, 2), jnp.float32),
                   jax.ShapeDtypeStruct((B, 256), jnp.float32)),
    )(flat, l1p, l1_b.reshape(1, -1), bn1_g.reshape(1, -1),
      bn1_b.reshape(1, -1), l2_w, l2_b.reshape(1, -1),
      bn2_g.reshape(1, -1), bn2_b.reshape(1, -1), l3_w, l3_b.reshape(1, -1))
    return probs, block


# speedup vs baseline: 1.7042x; 1.0311x over previous
import jax
import jax.numpy as jnp
from jax.experimental import pallas as pl

N = 116
NP = 128
F = 220
NHID = 20
NHEADS = 8
BS = 32


def _gat_kernel(xt_ref, w1t_ref, aw_ref, dw_ref, b1_ref,
                w2t_ref, as2_ref, ad2_ref, b2_ref, out_ref):
    row_i = jax.lax.broadcasted_iota(jnp.int32, (NP, NP), 0)
    col_j = jax.lax.broadcasted_iota(jnp.int32, (NP, NP), 1)
    eyef = jnp.where(row_i == col_j, 1.0, 0.0).astype(jnp.float32)
    ones_row = jnp.ones((1, NP), dtype=jnp.bfloat16)
    ones_fr = jnp.ones((1, F), dtype=jnp.float32)

    w1t = w1t_ref[...]
    aw = aw_ref[...]
    dw = dw_ref[...]
    w2t = w2t_ref[...]
    b1w = b1_ref[...] * jnp.ones((1, NP), dtype=jnp.float32)
    b2w = b2_ref[...] * jnp.ones((1, NP), dtype=jnp.float32)
    f32 = jnp.float32

    def leaky(v):
        return jnp.maximum(v, 0.2 * v)

    maskb, h1tb, a_srcc, a_dst, bound = [], [], [], [], []
    for s in range(BS):
        xt = xt_ref[s]
        mean = jnp.dot(ones_fr, xt, preferred_element_type=f32) * (1.0 / F)
        xct = xt - mean
        r = jnp.dot(ones_fr, xct * xct, preferred_element_type=f32)
        xnt = xct * jax.lax.rsqrt(jnp.maximum(r, 1e-30))
        corr = jax.lax.dot_general(xnt, xnt, (((0,), (0,)), ((), ())),
                                   preferred_element_type=f32)
        maskb.append(
            jnp.maximum(jnp.where(corr > 0.5, 1.0, 0.0), eyef)
            .astype(jnp.bfloat16))
        h1t = jnp.dot(w1t, xt, preferred_element_type=f32)
        h1tb.append(h1t.astype(jnp.bfloat16))
        asd = jnp.dot(aw, xt, preferred_element_type=f32)
        adst = jnp.dot(dw, xt, preferred_element_type=f32)
        a_srcc.append(jnp.transpose(asd))
        a_dst.append(adst)
        bound.append(leaky(jnp.max(asd, axis=1, keepdims=True) + adst))

    msgs = [[] for _ in range(BS)]
    for s in range(BS):
        for hd in range(NHEADS):
            src_c = a_srcc[s][:, hd:hd + 1]
            e = jnp.exp(leaky(src_c + a_dst[s][hd:hd + 1, :])
                        - bound[s][hd:hd + 1, :])
            eb = e.astype(jnp.bfloat16) * maskb[s]
            h1e = jnp.concatenate(
                [h1tb[s][hd * NHID:(hd + 1) * NHID, :], ones_row], axis=0)
            nd = jnp.dot(h1e, eb, preferred_element_type=f32)
            msgs[s].append(nd[:NHID, :]
                           * jax.lax.reciprocal(nd[NHID:NHID + 1, :]))

    for s in range(BS):
        out1 = jnp.concatenate(msgs[s], axis=0) + b1w
        out1 = jnp.where(out1 > 0, out1, jnp.exp(out1) - 1.0)
        h2t = jnp.dot(w2t, out1, preferred_element_type=f32)
        h2tb = h2t.astype(jnp.bfloat16)
        a2s = jnp.dot(as2_ref[...], h2t, preferred_element_type=f32)
        a2d = jnp.dot(ad2_ref[...], h2t, preferred_element_type=f32)
        bound2 = leaky(jnp.max(a2s, axis=1, keepdims=True) + a2d)
        e2 = jnp.exp(leaky(jnp.transpose(a2s) + a2d) - bound2)
        e2b = e2.astype(jnp.bfloat16) * maskb[s]
        h2e = jnp.concatenate([h2tb, ones_row], axis=0)
        nd2 = jnp.dot(h2e, e2b, preferred_element_type=f32)
        out_ref[s, :, :] = (nd2[:NHID, :]
                            * jax.lax.reciprocal(nd2[NHID:NHID + 1, :]) + b2w)


def _mlp_kernel(flat_ref, l1w_ref, l1b_ref, g1_ref, bt1_ref,
                l2w_ref, l2b_ref, g2_ref, bt2_ref, l3w_ref, l3b_ref,
                probs_ref, block_ref):
    inv = jnp.float32(1.0 / (1.0 + 1e-5) ** 0.5)
    h = jnp.dot(flat_ref[...], l1w_ref[...]) + l1b_ref[...]
    h = h * (g1_ref[...] * inv) + bt1_ref[...]
    blk = jnp.dot(h, l2w_ref[...]) + l2b_ref[...]
    blk = blk * (g2_ref[...] * inv) + bt2_ref[...]
    block_ref[...] = blk
    lg = jnp.dot(blk, l3w_ref[...]) + l3b_ref[...]
    m = jnp.max(lg, axis=1, keepdims=True)
    e = jnp.exp(lg - m)
    probs_ref[...] = e / jnp.sum(e, axis=1, keepdims=True)


@jax.jit
def kernel(input, W1, as1, ad1, b1, W2, as2, ad2, b2, l1_w, l1_b,
           bn1_g, bn1_b, l2_w, l2_b, bn2_g, bn2_b, l3_w, l3_b):
    B = input.shape[0]
    xpt = jnp.pad(input, ((0, 0), (0, NP - N), (0, 0))).transpose(0, 2, 1)
    eye8 = jnp.eye(NHEADS, dtype=jnp.float32)
    asr = (eye8[:, None, :] * as1[:, :, None]).reshape(
        NHEADS * NHID, NHEADS).T
    adt = (eye8[:, None, :] * ad1[:, :, None]).reshape(
        NHEADS * NHID, NHEADS).T
    aw = asr @ W1.T
    dw = adt @ W1.T
    l1p = l1_w.reshape(N, NHID, -1).transpose(1, 0, 2).reshape(N * NHID, -1)

    full = lambda shp: pl.BlockSpec(shp, lambda i: (0,) * len(shp))
    gat_t = pl.pallas_call(
        _gat_kernel,
        grid=(B // BS,),
        in_specs=[
            pl.BlockSpec((BS, F, NP), lambda i: (i, 0, 0)),
            full((NHEADS * NHID, F)),
            full((NHEADS, F)),
            full((NHEADS, F)),
            full((NHEADS * NHID, 1)),
            full((NHID, NHEADS * NHID)),
            full((1, NHID)),
            full((1, NHID)),
            full((NHID, 1)),
        ],
        out_specs=pl.BlockSpec((BS, NHID, NP), lambda i: (i, 0, 0)),
        out_shape=jax.ShapeDtypeStruct((B, NHID, NP), jnp.float32),
    )(xpt, W1.T, aw, dw, b1.reshape(-1, 1), W2.T,
      as2, ad2, b2.reshape(-1, 1))

    flat = gat_t[:, :, :N].reshape(B, NHID * N)
    probs, block = pl.pallas_call(
        _mlp_kernel,
        out_shape=(jax.ShapeDtypeStruct((B, 2), jnp.float32),
                   jax.ShapeDtypeStruct((B, 256), jnp.float32)),
    )(flat, l1p, l1_b.reshape(1, -1), bn1_g.reshape(1, -1),
      bn1_b.reshape(1, -1), l2_w, l2_b.reshape(1, -1),
      bn2_g.reshape(1, -1), bn2_b.reshape(1, -1), l3_w, l3_b.reshape(1, -1))
    return probs, block


# speedup vs baseline: 1.8631x; 1.0932x over previous
import jax
import jax.numpy as jnp
from jax.experimental import pallas as pl

N = 116
NP = 128
F = 220
NHID = 20
NHEADS = 8
BS = 32


def _gat_kernel(xt_ref, w1t_ref, aw_ref, dw_ref, b1_ref,
                w2t_ref, as2_ref, ad2_ref, b2_ref, out_ref):
    row_i = jax.lax.broadcasted_iota(jnp.int32, (N, N), 0)
    col_j = jax.lax.broadcasted_iota(jnp.int32, (N, N), 1)
    eyef = jnp.where(row_i == col_j, 1.0, 0.0).astype(jnp.float32)
    ones_row = jnp.ones((1, N), dtype=jnp.bfloat16)
    ones_fr = jnp.ones((1, F), dtype=jnp.float32)

    w1t = w1t_ref[...]
    aw = aw_ref[...]
    dw = dw_ref[...]
    w2t = w2t_ref[...]
    b1w = b1_ref[...] * jnp.ones((1, N), dtype=jnp.float32)
    b2w = b2_ref[...] * jnp.ones((1, N), dtype=jnp.float32)
    f32 = jnp.float32

    def leaky(v):
        return jnp.maximum(v, 0.2 * v)

    maskb, h1tb, a_srcc, a_dst, bound = [], [], [], [], []
    for s in range(BS):
        xt = jnp.transpose(xt_ref[s])
        mean = jnp.dot(ones_fr, xt, preferred_element_type=f32) * (1.0 / F)
        xct = xt - mean
        r = jnp.dot(ones_fr, xct * xct, preferred_element_type=f32)
        xnt = xct * jax.lax.rsqrt(jnp.maximum(r, 1e-30))
        corr = jax.lax.dot_general(xnt, xnt, (((0,), (0,)), ((), ())),
                                   preferred_element_type=f32)
        maskb.append(
            jnp.maximum(jnp.where(corr > 0.5, 1.0, 0.0), eyef)
            .astype(jnp.bfloat16))
        h1t = jnp.dot(w1t, xt, preferred_element_type=f32)
        h1tb.append(h1t.astype(jnp.bfloat16))
        asd = jnp.dot(aw, xt, preferred_element_type=f32)
        adst = jnp.dot(dw, xt, preferred_element_type=f32)
        a_srcc.append(jnp.transpose(asd))
        a_dst.append(adst)
        bound.append(leaky(jnp.max(asd, axis=1, keepdims=True) + adst))

    msgs = [[] for _ in range(BS)]
    for s in range(BS):
        for hd in range(NHEADS):
            src_c = a_srcc[s][:, hd:hd + 1]
            e = jnp.exp(leaky(src_c + a_dst[s][hd:hd + 1, :])
                        - bound[s][hd:hd + 1, :])
            eb = e.astype(jnp.bfloat16) * maskb[s]
            h1e = jnp.concatenate(
                [h1tb[s][hd * NHID:(hd + 1) * NHID, :], ones_row], axis=0)
            nd = jnp.dot(h1e, eb, preferred_element_type=f32)
            msgs[s].append(nd[:NHID, :]
                           * jax.lax.reciprocal(nd[NHID:NHID + 1, :]))

    for s in range(BS):
        out1 = jnp.concatenate(msgs[s], axis=0) + b1w
        out1 = jnp.where(out1 > 0, out1, jnp.exp(out1) - 1.0)
        h2t = jnp.dot(w2t, out1, preferred_element_type=f32)
        h2tb = h2t.astype(jnp.bfloat16)
        a2s = jnp.dot(as2_ref[...], h2t, preferred_element_type=f32)
        a2d = jnp.dot(ad2_ref[...], h2t, preferred_element_type=f32)
        bound2 = leaky(jnp.max(a2s, axis=1, keepdims=True) + a2d)
        e2 = jnp.exp(leaky(jnp.transpose(a2s) + a2d) - bound2)
        e2b = e2.astype(jnp.bfloat16) * maskb[s]
        h2e = jnp.concatenate([h2tb, ones_row], axis=0)
        nd2 = jnp.dot(h2e, e2b, preferred_element_type=f32)
        out_ref[s, :, :] = (nd2[:NHID, :]
                            * jax.lax.reciprocal(nd2[NHID:NHID + 1, :]) + b2w)


def _mlp_kernel(flat_ref, l1w_ref, l1b_ref, g1_ref, bt1_ref,
                l2w_ref, l2b_ref, g2_ref, bt2_ref, l3w_ref, l3b_ref,
                probs_ref, block_ref):
    inv = jnp.float32(1.0 / (1.0 + 1e-5) ** 0.5)
    h = jnp.dot(flat_ref[...], l1w_ref[...]) + l1b_ref[...]
    h = h * (g1_ref[...] * inv) + bt1_ref[...]
    blk = jnp.dot(h, l2w_ref[...]) + l2b_ref[...]
    blk = blk * (g2_ref[...] * inv) + bt2_ref[...]
    block_ref[...] = blk
    lg = jnp.dot(blk, l3w_ref[...]) + l3b_ref[...]
    m = jnp.max(lg, axis=1, keepdims=True)
    e = jnp.exp(lg - m)
    probs_ref[...] = e / jnp.sum(e, axis=1, keepdims=True)


@jax.jit
def kernel(input, W1, as1, ad1, b1, W2, as2, ad2, b2, l1_w, l1_b,
           bn1_g, bn1_b, l2_w, l2_b, bn2_g, bn2_b, l3_w, l3_b):
    B = input.shape[0]
    eye8 = jnp.eye(NHEADS, dtype=jnp.float32)
    asr = (eye8[:, None, :] * as1[:, :, None]).reshape(
        NHEADS * NHID, NHEADS).T
    adt = (eye8[:, None, :] * ad1[:, :, None]).reshape(
        NHEADS * NHID, NHEADS).T
    aw = asr @ W1.T
    dw = adt @ W1.T
    l1p = l1_w.reshape(N, NHID, -1).transpose(1, 0, 2).reshape(N * NHID, -1)

    full = lambda shp: pl.BlockSpec(shp, lambda i: (0,) * len(shp))
    gat_t = pl.pallas_call(
        _gat_kernel,
        grid=(B // BS,),
        in_specs=[
            pl.BlockSpec((BS, N, F), lambda i: (i, 0, 0)),
            full((NHEADS * NHID, F)),
            full((NHEADS, F)),
            full((NHEADS, F)),
            full((NHEADS * NHID, 1)),
            full((NHID, NHEADS * NHID)),
            full((1, NHID)),
            full((1, NHID)),
            full((NHID, 1)),
        ],
        out_specs=pl.BlockSpec((BS, NHID, N), lambda i: (i, 0, 0)),
        out_shape=jax.ShapeDtypeStruct((B, NHID, N), jnp.float32),
    )(input, W1.T, aw, dw, b1.reshape(-1, 1), W2.T,
      as2, ad2, b2.reshape(-1, 1))

    flat = gat_t.reshape(B, NHID * N)
    probs, block = pl.pallas_call(
        _mlp_kernel,
        out_shape=(jax.ShapeDtypeStruct((B, 2), jnp.float32),
                   jax.ShapeDtypeStruct((B, 256), jnp.float32)),
    )(flat, l1p, l1_b.reshape(1, -1), bn1_g.reshape(1, -1),
      bn1_b.reshape(1, -1), l2_w, l2_b.reshape(1, -1),
      bn2_g.reshape(1, -1), bn2_b.reshape(1, -1), l3_w, l3_b.reshape(1, -1))
    return probs, block


# speedup vs baseline: 1.8879x; 1.0134x over previous
import jax
import jax.numpy as jnp
from jax.experimental import pallas as pl

N = 116
NP = 128
F = 220
NHID = 20
NHEADS = 8
BS = 16


def _gat_kernel(xt_ref, w1t_ref, aw_ref, dw_ref, b1_ref,
                w2t_ref, as2_ref, ad2_ref, b2_ref, out_ref):
    row_i = jax.lax.broadcasted_iota(jnp.int32, (N, N), 0)
    col_j = jax.lax.broadcasted_iota(jnp.int32, (N, N), 1)
    eyef = jnp.where(row_i == col_j, 1.0, 0.0).astype(jnp.float32)
    ones_row = jnp.ones((1, N), dtype=jnp.bfloat16)
    ones_fr = jnp.ones((1, F), dtype=jnp.float32)

    w1t = w1t_ref[...]
    aw = aw_ref[...]
    dw = dw_ref[...]
    w2t = w2t_ref[...]
    b1w = b1_ref[...] * jnp.ones((1, N), dtype=jnp.float32)
    b2w = b2_ref[...] * jnp.ones((1, N), dtype=jnp.float32)
    f32 = jnp.float32

    def leaky(v):
        return jnp.maximum(v, 0.2 * v)

    maskb, h1tb, a_srcc, a_dst, bound = [], [], [], [], []
    for s in range(BS):
        xt = jnp.transpose(xt_ref[s])
        mean = jnp.dot(ones_fr, xt, preferred_element_type=f32) * (1.0 / F)
        xct = xt - mean
        r = jnp.dot(ones_fr, xct * xct, preferred_element_type=f32)
        xnt = xct * jax.lax.rsqrt(jnp.maximum(r, 1e-30))
        corr = jax.lax.dot_general(xnt, xnt, (((0,), (0,)), ((), ())),
                                   preferred_element_type=f32)
        maskb.append(
            jnp.maximum(jnp.where(corr > 0.5, 1.0, 0.0), eyef)
            .astype(jnp.bfloat16))
        h1t = jnp.dot(w1t, xt, preferred_element_type=f32)
        h1tb.append(h1t.astype(jnp.bfloat16))
        asd = jnp.dot(aw, xt, preferred_element_type=f32)
        adst = jnp.dot(dw, xt, preferred_element_type=f32)
        a_srcc.append(jnp.transpose(asd))
        a_dst.append(adst)
        bound.append(leaky(jnp.max(asd, axis=1, keepdims=True) + adst))

    msgs = [[] for _ in range(BS)]
    for s in range(BS):
        for hd in range(NHEADS):
            src_c = a_srcc[s][:, hd:hd + 1]
            e = jnp.exp(leaky(src_c + a_dst[s][hd:hd + 1, :])
                        - bound[s][hd:hd + 1, :])
            eb = e.astype(jnp.bfloat16) * maskb[s]
            h1e = jnp.concatenate(
                [h1tb[s][hd * NHID:(hd + 1) * NHID, :], ones_row], axis=0)
            nd = jnp.dot(h1e, eb, preferred_element_type=f32)
            msgs[s].append(nd[:NHID, :]
                           * jax.lax.reciprocal(nd[NHID:NHID + 1, :]))

    for s in range(BS):
        out1 = jnp.concatenate(msgs[s], axis=0) + b1w
        out1 = jnp.where(out1 > 0, out1, jnp.exp(out1) - 1.0)
        h2t = jnp.dot(w2t, out1, preferred_element_type=f32)
        h2tb = h2t.astype(jnp.bfloat16)
        a2s = jnp.dot(as2_ref[...], h2t, preferred_element_type=f32)
        a2d = jnp.dot(ad2_ref[...], h2t, preferred_element_type=f32)
        bound2 = leaky(jnp.max(a2s, axis=1, keepdims=True) + a2d)
        e2 = jnp.exp(leaky(jnp.transpose(a2s) + a2d) - bound2)
        e2b = e2.astype(jnp.bfloat16) * maskb[s]
        h2e = jnp.concatenate([h2tb, ones_row], axis=0)
        nd2 = jnp.dot(h2e, e2b, preferred_element_type=f32)
        out_ref[s, :, :] = (nd2[:NHID, :]
                            * jax.lax.reciprocal(nd2[NHID:NHID + 1, :]) + b2w)


def _mlp_kernel(flat_ref, l1w_ref, l1b_ref, g1_ref, bt1_ref,
                l2w_ref, l2b_ref, g2_ref, bt2_ref, l3w_ref, l3b_ref,
                probs_ref, block_ref):
    inv = jnp.float32(1.0 / (1.0 + 1e-5) ** 0.5)
    h = jnp.dot(flat_ref[...], l1w_ref[...]) + l1b_ref[...]
    h = h * (g1_ref[...] * inv) + bt1_ref[...]
    blk = jnp.dot(h, l2w_ref[...]) + l2b_ref[...]
    blk = blk * (g2_ref[...] * inv) + bt2_ref[...]
    block_ref[...] = blk
    lg = jnp.dot(blk, l3w_ref[...]) + l3b_ref[...]
    m = jnp.max(lg, axis=1, keepdims=True)
    e = jnp.exp(lg - m)
    probs_ref[...] = e / jnp.sum(e, axis=1, keepdims=True)


@jax.jit
def kernel(input, W1, as1, ad1, b1, W2, as2, ad2, b2, l1_w, l1_b,
           bn1_g, bn1_b, l2_w, l2_b, bn2_g, bn2_b, l3_w, l3_b):
    B = input.shape[0]
    eye8 = jnp.eye(NHEADS, dtype=jnp.float32)
    asr = (eye8[:, None, :] * as1[:, :, None]).reshape(
        NHEADS * NHID, NHEADS).T
    adt = (eye8[:, None, :] * ad1[:, :, None]).reshape(
        NHEADS * NHID, NHEADS).T
    aw = asr @ W1.T
    dw = adt @ W1.T
    l1p = l1_w.reshape(N, NHID, -1).transpose(1, 0, 2).reshape(N * NHID, -1)

    full = lambda shp: pl.BlockSpec(shp, lambda i: (0,) * len(shp))
    gat_t = pl.pallas_call(
        _gat_kernel,
        grid=(B // BS,),
        in_specs=[
            pl.BlockSpec((BS, N, F), lambda i: (i, 0, 0)),
            full((NHEADS * NHID, F)),
            full((NHEADS, F)),
            full((NHEADS, F)),
            full((NHEADS * NHID, 1)),
            full((NHID, NHEADS * NHID)),
            full((1, NHID)),
            full((1, NHID)),
            full((NHID, 1)),
        ],
        out_specs=pl.BlockSpec((BS, NHID, N), lambda i: (i, 0, 0)),
        out_shape=jax.ShapeDtypeStruct((B, NHID, N), jnp.float32),
    )(input, W1.T, aw, dw, b1.reshape(-1, 1), W2.T,
      as2, ad2, b2.reshape(-1, 1))

    flat = gat_t.reshape(B, NHID * N)
    probs, block = pl.pallas_call(
        _mlp_kernel,
        out_shape=(jax.ShapeDtypeStruct((B, 2), jnp.float32),
                   jax.ShapeDtypeStruct((B, 256), jnp.float32)),
    )(flat, l1p, l1_b.reshape(1, -1), bn1_g.reshape(1, -1),
      bn1_b.reshape(1, -1), l2_w, l2_b.reshape(1, -1),
      bn2_g.reshape(1, -1), bn2_b.reshape(1, -1), l3_w, l3_b.reshape(1, -1))
    return probs, block


# speedup vs baseline: 1.9176x; 1.0157x over previous
import jax
import jax.numpy as jnp
from jax.experimental import pallas as pl

N = 116
NP = 128
F = 220
NHID = 20
NHEADS = 8
BS = 16


def _gat_kernel(xt_ref, w1t_ref, aw_ref, dw_ref, b1_ref,
                w2t_ref, as2_ref, ad2_ref, b2_ref, out_ref):
    row_i = jax.lax.broadcasted_iota(jnp.int32, (N, N), 0)
    col_j = jax.lax.broadcasted_iota(jnp.int32, (N, N), 1)
    eyef = jnp.where(row_i == col_j, 1.0, 0.0).astype(jnp.float32)
    ones_row = jnp.ones((1, N), dtype=jnp.bfloat16)
    ones_fr = jnp.ones((1, F), dtype=jnp.float32)

    w1t = w1t_ref[...]
    aw = aw_ref[...]
    dw = dw_ref[...]
    w2t = w2t_ref[...]
    b1w = b1_ref[...] * jnp.ones((1, N), dtype=jnp.float32)
    b2w = b2_ref[...] * jnp.ones((1, N), dtype=jnp.float32)
    f32 = jnp.float32

    def leaky(v):
        return jnp.maximum(v, 0.2 * v)

    maskb, h1tb, a_srcc, a_dst, bound = [], [], [], [], []
    for s in range(BS):
        xt = jnp.transpose(xt_ref[s])
        mean = jnp.dot(ones_fr, xt, preferred_element_type=f32) * (1.0 / F)
        xct = xt - mean
        r = jnp.dot(ones_fr, xct * xct, preferred_element_type=f32)
        xnt = xct * jax.lax.rsqrt(jnp.maximum(r, 1e-30))
        corr = jax.lax.dot_general(xnt, xnt, (((0,), (0,)), ((), ())),
                                   preferred_element_type=f32)
        maskb.append(
            jnp.maximum(jnp.where(corr > 0.5, 1.0, 0.0), eyef)
            .astype(jnp.bfloat16))
        h1t = jnp.dot(w1t, xt, preferred_element_type=f32)
        h1tb.append(h1t.astype(jnp.bfloat16))
        asd = jnp.dot(aw, xt, preferred_element_type=f32)
        adst = jnp.dot(dw, xt, preferred_element_type=f32)
        a_srcc.append(jnp.transpose(asd).astype(jnp.bfloat16))
        a_dst.append(adst.astype(jnp.bfloat16))
        bound.append(
            leaky(jnp.max(asd, axis=1, keepdims=True) + adst)
            .astype(jnp.bfloat16))

    msgs = [[] for _ in range(BS)]
    for s in range(BS):
        for hd in range(NHEADS):
            src_c = a_srcc[s][:, hd:hd + 1]
            e = jnp.exp(leaky(src_c + a_dst[s][hd:hd + 1, :])
                        - bound[s][hd:hd + 1, :])
            eb = e * maskb[s]
            h1e = jnp.concatenate(
                [h1tb[s][hd * NHID:(hd + 1) * NHID, :], ones_row], axis=0)
            nd = jnp.dot(h1e, eb, preferred_element_type=f32)
            msgs[s].append(nd[:NHID, :]
                           * jax.lax.reciprocal(nd[NHID:NHID + 1, :]))

    for s in range(BS):
        out1 = jnp.concatenate(msgs[s], axis=0) + b1w
        out1 = jnp.where(out1 > 0, out1, jnp.exp(out1) - 1.0)
        h2t = jnp.dot(w2t, out1, preferred_element_type=f32)
        h2tb = h2t.astype(jnp.bfloat16)
        a2s = jnp.dot(as2_ref[...], h2t, preferred_element_type=f32)
        a2d = jnp.dot(ad2_ref[...], h2t, preferred_element_type=f32)
        bound2 = leaky(jnp.max(a2s, axis=1, keepdims=True) + a2d)
        e2 = jnp.exp(leaky(jnp.transpose(a2s) + a2d) - bound2)
        e2b = e2.astype(jnp.bfloat16) * maskb[s]
        h2e = jnp.concatenate([h2tb, ones_row], axis=0)
        nd2 = jnp.dot(h2e, e2b, preferred_element_type=f32)
        out_ref[s, :, :] = (nd2[:NHID, :]
                            * jax.lax.reciprocal(nd2[NHID:NHID + 1, :]) + b2w)


def _mlp_kernel(flat_ref, l1w_ref, l1b_ref, g1_ref, bt1_ref,
                l2w_ref, l2b_ref, g2_ref, bt2_ref, l3w_ref, l3b_ref,
                probs_ref, block_ref):
    inv = jnp.float32(1.0 / (1.0 + 1e-5) ** 0.5)
    h = jnp.dot(flat_ref[...], l1w_ref[...]) + l1b_ref[...]
    h = h * (g1_ref[...] * inv) + bt1_ref[...]
    blk = jnp.dot(h, l2w_ref[...]) + l2b_ref[...]
    blk = blk * (g2_ref[...] * inv) + bt2_ref[...]
    block_ref[...] = blk
    lg = jnp.dot(blk, l3w_ref[...]) + l3b_ref[...]
    m = jnp.max(lg, axis=1, keepdims=True)
    e = jnp.exp(lg - m)
    probs_ref[...] = e / jnp.sum(e, axis=1, keepdims=True)


@jax.jit
def kernel(input, W1, as1, ad1, b1, W2, as2, ad2, b2, l1_w, l1_b,
           bn1_g, bn1_b, l2_w, l2_b, bn2_g, bn2_b, l3_w, l3_b):
    B = input.shape[0]
    eye8 = jnp.eye(NHEADS, dtype=jnp.float32)
    asr = (eye8[:, None, :] * as1[:, :, None]).reshape(
        NHEADS * NHID, NHEADS).T
    adt = (eye8[:, None, :] * ad1[:, :, None]).reshape(
        NHEADS * NHID, NHEADS).T
    aw = asr @ W1.T
    dw = adt @ W1.T
    l1p = l1_w.reshape(N, NHID, -1).transpose(1, 0, 2).reshape(N * NHID, -1)

    full = lambda shp: pl.BlockSpec(shp, lambda i: (0,) * len(shp))
    gat_t = pl.pallas_call(
        _gat_kernel,
        grid=(B // BS,),
        in_specs=[
            pl.BlockSpec((BS, N, F), lambda i: (i, 0, 0)),
            full((NHEADS * NHID, F)),
            full((NHEADS, F)),
            full((NHEADS, F)),
            full((NHEADS * NHID, 1)),
            full((NHID, NHEADS * NHID)),
            full((1, NHID)),
            full((1, NHID)),
            full((NHID, 1)),
        ],
        out_specs=pl.BlockSpec((BS, NHID, N), lambda i: (i, 0, 0)),
        out_shape=jax.ShapeDtypeStruct((B, NHID, N), jnp.float32),
    )(input, W1.T, aw, dw, b1.reshape(-1, 1), W2.T,
      as2, ad2, b2.reshape(-1, 1))

    flat = gat_t.reshape(B, NHID * N)
    probs, block = pl.pallas_call(
        _mlp_kernel,
        out_shape=(jax.ShapeDtypeStruct((B, 2), jnp.float32),
                   jax.ShapeDtypeStruct((B, 256), jnp.float32)),
    )(flat, l1p, l1_b.reshape(1, -1), bn1_g.reshape(1, -1),
      bn1_b.reshape(1, -1), l2_w, l2_b.reshape(1, -1),
      bn2_g.reshape(1, -1), bn2_b.reshape(1, -1), l3_w, l3_b.reshape(1, -1))
    return probs, block


# speedup vs baseline: 2.0520x; 1.0701x over previous
import jax
import jax.numpy as jnp
from jax.experimental import pallas as pl

N = 116
NP = 128
F = 220
NHID = 20
NHEADS = 8
BS = 16


def _gat_kernel(xt_ref, w1t_ref, aw_ref, dw_ref, b1_ref,
                w2t_ref, as2_ref, ad2_ref, b2_ref, out_ref):
    row_i = jax.lax.broadcasted_iota(jnp.int32, (N, N), 0)
    col_j = jax.lax.broadcasted_iota(jnp.int32, (N, N), 1)
    eyef = jnp.where(row_i == col_j, 1.0, 0.0).astype(jnp.float32)
    ones_row = jnp.ones((1, N), dtype=jnp.bfloat16)
    ones_fr = jnp.ones((1, F), dtype=jnp.float32)

    w1t = w1t_ref[...]
    aw = aw_ref[...]
    dw = dw_ref[...]
    w2t = w2t_ref[...]
    b1w = b1_ref[...] * jnp.ones((1, N), dtype=jnp.float32)
    b2w = b2_ref[...] * jnp.ones((1, N), dtype=jnp.float32)
    f32 = jnp.float32

    def leaky(v):
        return jnp.maximum(v, 0.2 * v)

    maskb, h1tb, a_srcc, a_dst, bound = [], [], [], [], []
    for s in range(BS):
        xt = jnp.transpose(xt_ref[s])
        mean = jnp.dot(ones_fr, xt, preferred_element_type=f32) * (1.0 / F)
        xct = xt - mean
        r = jnp.dot(ones_fr, xct * xct, preferred_element_type=f32)
        xnt = xct * jax.lax.rsqrt(jnp.maximum(r, 1e-30))
        corr = jax.lax.dot_general(xnt, xnt, (((0,), (0,)), ((), ())),
                                   preferred_element_type=f32)
        maskb.append(
            jnp.maximum(jnp.where(corr > 0.5, 1.0, 0.0), eyef)
            .astype(jnp.bfloat16))
        h1t = jnp.dot(w1t, xt, preferred_element_type=f32)
        h1tb.append(h1t.astype(jnp.bfloat16))
        asd = jnp.dot(aw, xt, preferred_element_type=f32)
        adst = jnp.dot(dw, xt, preferred_element_type=f32)
        a_srcc.append(jnp.transpose(asd).astype(jnp.bfloat16))
        a_dst.append(adst.astype(jnp.bfloat16))
        bound.append(
            leaky(jnp.max(asd, axis=1, keepdims=True) + adst)
            .astype(jnp.bfloat16))

    msgs = [[] for _ in range(BS)]
    for s in range(BS):
        for hd in range(NHEADS):
            src_c = a_srcc[s][:, hd:hd + 1]
            e = jnp.exp(leaky(src_c + a_dst[s][hd:hd + 1, :])
                        - bound[s][hd:hd + 1, :])
            eb = e * maskb[s]
            h1e = jnp.concatenate(
                [h1tb[s][hd * NHID:(hd + 1) * NHID, :], ones_row], axis=0)
            nd = jnp.dot(h1e, eb, preferred_element_type=f32)
            msgs[s].append(nd[:NHID, :]
                           * jax.lax.reciprocal(nd[NHID:NHID + 1, :]))

    for s in range(BS):
        out1 = jnp.concatenate(msgs[s], axis=0) + b1w
        out1 = jnp.where(out1 > 0, out1, jnp.exp(out1) - 1.0)
        h2t = jnp.dot(w2t, out1, preferred_element_type=f32)
        h2tb = h2t.astype(jnp.bfloat16)
        a2s = jnp.dot(as2_ref[...], h2t, preferred_element_type=f32)
        a2d = jnp.dot(ad2_ref[...], h2t, preferred_element_type=f32)
        bound2 = leaky(jnp.max(a2s, axis=1, keepdims=True) + a2d)
        e2 = jnp.exp(leaky(jnp.transpose(a2s) + a2d) - bound2)
        e2b = e2.astype(jnp.bfloat16) * maskb[s]
        h2e = jnp.concatenate([h2tb, ones_row], axis=0)
        nd2 = jnp.dot(h2e, e2b, preferred_element_type=f32)
        out_ref[s, :, :] = (nd2[:NHID, :]
                            * jax.lax.reciprocal(nd2[NHID:NHID + 1, :]) + b2w)


def _mlp_kernel(flat_ref, l1w_ref, l1b_ref, g1_ref, bt1_ref,
                l2w_ref, l2b_ref, g2_ref, bt2_ref, l3w_ref, l3b_ref,
                probs_ref, block_ref):
    inv = jnp.float32(1.0 / (1.0 + 1e-5) ** 0.5)
    h = jnp.dot(flat_ref[...], l1w_ref[...]) + l1b_ref[...]
    h = h * (g1_ref[...] * inv) + bt1_ref[...]
    blk = jnp.dot(h, l2w_ref[...]) + l2b_ref[...]
    blk = blk * (g2_ref[...] * inv) + bt2_ref[...]
    block_ref[...] = blk
    lg = jnp.dot(blk, l3w_ref[...]) + l3b_ref[...]
    m = jnp.max(lg, axis=1, keepdims=True)
    e = jnp.exp(lg - m)
    probs_ref[...] = e / jnp.sum(e, axis=1, keepdims=True)


@jax.jit
def kernel(input, W1, as1, ad1, b1, W2, as2, ad2, b2, l1_w, l1_b,
           bn1_g, bn1_b, l2_w, l2_b, bn2_g, bn2_b, l3_w, l3_b):
    B = input.shape[0]
    eye8 = jnp.eye(NHEADS, dtype=jnp.float32)
    asr = (eye8[:, None, :] * as1[:, :, None]).reshape(
        NHEADS * NHID, NHEADS).T
    adt = (eye8[:, None, :] * ad1[:, :, None]).reshape(
        NHEADS * NHID, NHEADS).T
    aw = asr @ W1.T
    dw = adt @ W1.T

    full = lambda shp: pl.BlockSpec(shp, lambda i: (0,) * len(shp))
    gat_t = pl.pallas_call(
        _gat_kernel,
        grid=(B // BS,),
        in_specs=[
            pl.BlockSpec((BS, N, F), lambda i: (i, 0, 0)),
            full((NHEADS * NHID, F)),
            full((NHEADS, F)),
            full((NHEADS, F)),
            full((NHEADS * NHID, 1)),
            full((NHID, NHEADS * NHID)),
            full((1, NHID)),
            full((1, NHID)),
            full((NHID, 1)),
        ],
        out_specs=pl.BlockSpec((BS, NHID, N), lambda i: (i, 0, 0)),
        out_shape=jax.ShapeDtypeStruct((B, NHID, N), jnp.float32),
    )(input, W1.T, aw, dw, b1.reshape(-1, 1), W2.T,
      as2, ad2, b2.reshape(-1, 1))

    flat = gat_t.transpose(0, 2, 1).reshape(B, N * NHID)
    probs, block = pl.pallas_call(
        _mlp_kernel,
        out_shape=(jax.ShapeDtypeStruct((B, 2), jnp.float32),
                   jax.ShapeDtypeStruct((B, 256), jnp.float32)),
    )(flat, l1_w, l1_b.reshape(1, -1), bn1_g.reshape(1, -1),
      bn1_b.reshape(1, -1), l2_w, l2_b.reshape(1, -1),
      bn2_g.reshape(1, -1), bn2_b.reshape(1, -1), l3_w, l3_b.reshape(1, -1))
    return probs, block
